# baseline (device time: 71503 ns/iter reference)
import jax
import jax.numpy as jnp
from jax import lax
from jax.experimental import pallas as pl
from jax.experimental.pallas import tpu as pltpu

N_DEV = 4
B, SQ, HQ, DH = 2, 512, 8, 64
SKV_LOC = 512
D_MODEL = 768
BH = B * HQ
BLK = 64
CHUNK = SQ // N_DEV
NPAIR = BH // 2


def kernel(x, Wq, K_ext, V_ext, Wo):
    K2 = K_ext.transpose(0, 2, 1, 3).reshape(BH, SKV_LOC, DH)
    V2 = V_ext.transpose(0, 2, 1, 3).reshape(BH, SKV_LOC, DH)
    Wq2 = Wq.reshape(D_MODEL, HQ, DH).transpose(1, 0, 2)

    def body(x_ref, wq_ref, k_ref, v_ref, wo_ref, out_ref,
             rs_o, rs_l, p_o, p_l, ctx_s, ag,
             rso_send, rso_recv, rsl_send, rsl_recv, ag_send, ag_recv):
        my = lax.axis_index("i")
        left = lax.rem(my + N_DEV - 1, N_DEV)
        right = lax.rem(my + 1, N_DEV)

        barrier = pltpu.get_barrier_semaphore()
        for nbr in (left, right):
            pl.semaphore_signal(barrier, inc=1, device_id=(nbr,),
                                device_id_type=pl.DeviceIdType.MESH)
        pl.semaphore_wait(barrier, 2)

        def compute_partial(c, o_dst, l_dst, o_slot, l_slot):
            qb = (lax.broadcasted_iota(jnp.int32, (CHUNK, SKV_LOC), 0)
                  + c * CHUNK) // BLK
            kb = (lax.broadcasted_iota(jnp.int32, (CHUNK, SKV_LOC), 1) // BLK
                  + my * (SKV_LOC // BLK))
            m = (qb == kb) | (kb == 0) | (((qb + kb) % 3) == 0)
            for b in range(B):
                xb = x_ref[b, pl.ds(c * CHUNK, CHUNK), :]
                for h in range(HQ):
                    bh = b * HQ + h
                    pair, off = bh // 2, (bh % 2) * DH
                    q = jnp.dot(xb, wq_ref[h],
                                preferred_element_type=jnp.float32)
                    s = lax.dot_general(
                        q, k_ref[bh], (((1,), (1,)), ((), ())),
                        preferred_element_type=jnp.float32,
                    ) * 0.125
                    w = jnp.where(m, jnp.exp(s), 0.0)
                    o = jnp.dot(w, v_ref[bh],
                                preferred_element_type=jnp.float32)
                    o_dst[o_slot, pair, :, off:off + DH] = o
                    l_dst[l_slot, bh, :] = jnp.sum(w, axis=1)

        compute_partial(lax.rem(my + N_DEV - 1, N_DEV), rs_o, rs_l, 0, 0)
        for t in range(N_DEV - 1):
            ss, rs = t % 2, (t + 1) % 2
            ro = pltpu.make_async_remote_copy(
                src_ref=rs_o.at[ss], dst_ref=rs_o.at[rs],
                send_sem=rso_send.at[ss], recv_sem=rso_recv.at[rs],
                device_id=(right,), device_id_type=pl.DeviceIdType.MESH)
            rl = pltpu.make_async_remote_copy(
                src_ref=rs_l.at[ss], dst_ref=rs_l.at[rs],
                send_sem=rsl_send.at[ss], recv_sem=rsl_recv.at[rs],
                device_id=(right,), device_id_type=pl.DeviceIdType.MESH)
            ro.start()
            rl.start()
            compute_partial(lax.rem(my + 2 * N_DEV - 2 - t, N_DEV),
                            p_o, p_l, 0, 0)
            ro.wait()
            rl.wait()
            rs_o[rs] = rs_o[rs] + p_o[0]
            rs_l[rs] = rs_l[rs] + p_l[0]
        fin = (N_DEV - 1) % 2

        for b in range(B):
            for h in range(HQ):
                bh = b * HQ + h
                pair, off = bh // 2, (bh % 2) * DH
                lrow = rs_l[fin, bh, :]
                ctx_s[b, :, h * DH:(h + 1) * DH] = (
                    rs_o[fin, pair, :, off:off + DH] / lrow[:, None])
        for b in range(B):
            oc = jnp.dot(ctx_s[b], wo_ref[:, :],
                         preferred_element_type=jnp.float32)
            ag[0, b] = oc
            out_ref[b, pl.ds(my * CHUNK, CHUNK), :] = oc

        for t in range(N_DEV - 1):
            ss, rs = t % 2, (t + 1) % 2
            r = pltpu.make_async_remote_copy(
                src_ref=ag.at[ss], dst_ref=ag.at[rs],
                send_sem=ag_send.at[ss], recv_sem=ag_recv.at[rs],
                device_id=(right,), device_id_type=pl.DeviceIdType.MESH)
            r.start()
            r.wait()
            origin = lax.rem(my + 2 * N_DEV - 1 - t, N_DEV)
            for b in range(B):
                out_ref[b, pl.ds(origin * CHUNK, CHUNK), :] = ag[rs, b]

    return pl.pallas_call(
        body,
        out_shape=jax.ShapeDtypeStruct((B, SQ, D_MODEL), jnp.float32),
        in_specs=[pl.BlockSpec(memory_space=pltpu.VMEM)] * 5,
        out_specs=pl.BlockSpec(memory_space=pltpu.VMEM),
        scratch_shapes=[
            pltpu.VMEM((2, NPAIR, CHUNK, 2 * DH), jnp.float32),
            pltpu.VMEM((2, BH, CHUNK), jnp.float32),
            pltpu.VMEM((1, NPAIR, CHUNK, 2 * DH), jnp.float32),
            pltpu.VMEM((1, BH, CHUNK), jnp.float32),
            pltpu.VMEM((B, CHUNK, HQ * DH), jnp.float32),
            pltpu.VMEM((2, B, CHUNK, D_MODEL), jnp.float32),
            pltpu.SemaphoreType.DMA((2,)),
            pltpu.SemaphoreType.DMA((2,)),
            pltpu.SemaphoreType.DMA((2,)),
            pltpu.SemaphoreType.DMA((2,)),
            pltpu.SemaphoreType.DMA((2,)),
            pltpu.SemaphoreType.DMA((2,)),
        ],
        compiler_params=pltpu.CompilerParams(collective_id=0),
    )(x, Wq2, K2, V2, Wo)


# device time: 44472 ns/iter; 1.6078x vs baseline; 1.6078x over previous
import jax
import jax.numpy as jnp
from jax import lax
from jax.experimental import pallas as pl
from jax.experimental.pallas import tpu as pltpu

N_DEV = 4
B, SQ, HQ, DH = 2, 512, 8, 64
SKV_LOC = 512
D_MODEL = 768
BH = B * HQ
BLK = 64
CHUNK = SQ // N_DEV
NPAIR = BH // 2
HPAIR = HQ // 2


def kernel(x, Wq, K_ext, V_ext, Wo):
    K2 = K_ext.transpose(0, 2, 1, 3).reshape(BH, SKV_LOC, DH)
    V2 = V_ext.transpose(0, 2, 1, 3).reshape(BH, SKV_LOC, DH)
    Wq2 = Wq.reshape(D_MODEL, HPAIR, 2 * DH).transpose(1, 0, 2)

    def body(x_ref, wq_ref, k_ref, v_ref, wo_ref, out_ref,
             rs_o, rs_l, p_o, p_l, ctx_s, ag,
             rso_send, rso_recv, rsl_send, rsl_recv, ag_send, ag_recv):
        my = lax.axis_index("i")
        left = lax.rem(my + N_DEV - 1, N_DEV)
        right = lax.rem(my + 1, N_DEV)

        barrier = pltpu.get_barrier_semaphore()
        for nbr in (left, right):
            pl.semaphore_signal(barrier, inc=1, device_id=(nbr,),
                                device_id_type=pl.DeviceIdType.MESH)
        pl.semaphore_wait(barrier, 2)

        def compute_partial(c, o_dst, l_dst, o_slot, l_slot, o_dtype):
            qb = (lax.broadcasted_iota(jnp.int32, (CHUNK, SKV_LOC), 0)
                  + c * CHUNK) // BLK
            kb = (lax.broadcasted_iota(jnp.int32, (CHUNK, SKV_LOC), 1) // BLK
                  + my * (SKV_LOC // BLK))
            m = (qb == kb) | (kb == 0) | (((qb + kb) % 3) == 0)
            for b in range(B):
                xb = x_ref[b, pl.ds(c * CHUNK, CHUNK), :]
                for hp in range(HPAIR):
                    q2 = jnp.dot(xb, wq_ref[hp],
                                 preferred_element_type=jnp.float32)
                    for sub in range(2):
                        h = 2 * hp + sub
                        bh = b * HQ + h
                        pair, off = bh // 2, sub * DH
                        q = q2[:, off:off + DH]
                        s = lax.dot_general(
                            q, k_ref[bh], (((1,), (1,)), ((), ())),
                            preferred_element_type=jnp.float32,
                        ) * 0.125
                        w = jnp.where(m, jnp.exp(s), 0.0)
                        o = jnp.dot(w, v_ref[bh],
                                    preferred_element_type=jnp.float32)
                        o_dst[o_slot, pair, :, off:off + DH] = o.astype(o_dtype)
                        l_dst[l_slot, bh, :] = jnp.sum(w, axis=1)

        compute_partial(lax.rem(my + N_DEV - 1, N_DEV), rs_o, rs_l, 0, 0,
                        jnp.bfloat16)
        for t in range(N_DEV - 1):
            ss, rs = t % 2, (t + 1) % 2
            ro = pltpu.make_async_remote_copy(
                src_ref=rs_o.at[ss], dst_ref=rs_o.at[rs],
                send_sem=rso_send.at[ss], recv_sem=rso_recv.at[rs],
                device_id=(right,), device_id_type=pl.DeviceIdType.MESH)
            rl = pltpu.make_async_remote_copy(
                src_ref=rs_l.at[ss], dst_ref=rs_l.at[rs],
                send_sem=rsl_send.at[ss], recv_sem=rsl_recv.at[rs],
                device_id=(right,), device_id_type=pl.DeviceIdType.MESH)
            ro.start()
            rl.start()
            compute_partial(lax.rem(my + 2 * N_DEV - 2 - t, N_DEV),
                            p_o, p_l, 0, 0, jnp.float32)
            ro.wait()
            rl.wait()
            rs_o[rs] = (rs_o[rs].astype(jnp.float32)
                        + p_o[0]).astype(jnp.bfloat16)
            rs_l[rs] = rs_l[rs] + p_l[0]
        fin = (N_DEV - 1) % 2

        for b in range(B):
            for h in range(HQ):
                bh = b * HQ + h
                pair, off = bh // 2, (bh % 2) * DH
                lrow = rs_l[fin, bh, :]
                ctx_s[b, :, h * DH:(h + 1) * DH] = (
                    rs_o[fin, pair, :, off:off + DH].astype(jnp.float32)
                    / lrow[:, None])
        for b in range(B):
            oc = jnp.dot(ctx_s[b], wo_ref[:, :],
                         preferred_element_type=jnp.float32)
            ag[0, b] = oc.astype(jnp.bfloat16)
            out_ref[b, pl.ds(my * CHUNK, CHUNK), :] = oc

        a_r = pltpu.make_async_remote_copy(
            src_ref=ag.at[0], dst_ref=ag.at[1],
            send_sem=ag_send.at[0], recv_sem=ag_recv.at[1],
            device_id=(right,), device_id_type=pl.DeviceIdType.MESH)
        a_l = pltpu.make_async_remote_copy(
            src_ref=ag.at[0], dst_ref=ag.at[2],
            send_sem=ag_send.at[1], recv_sem=ag_recv.at[2],
            device_id=(left,), device_id_type=pl.DeviceIdType.MESH)
        a_r.start()
        a_l.start()
        a_r.wait()
        a_l.wait()
        fwd = pltpu.make_async_remote_copy(
            src_ref=ag.at[1], dst_ref=ag.at[3],
            send_sem=ag_send.at[2], recv_sem=ag_recv.at[3],
            device_id=(right,), device_id_type=pl.DeviceIdType.MESH)
        fwd.start()
        for slot, origin in ((1, left), (2, right)):
            for b in range(B):
                out_ref[b, pl.ds(origin * CHUNK, CHUNK), :] = (
                    ag[slot, b].astype(jnp.float32))
        fwd.wait()
        opp = lax.rem(my + 2, N_DEV)
        for b in range(B):
            out_ref[b, pl.ds(opp * CHUNK, CHUNK), :] = (
                ag[3, b].astype(jnp.float32))

    return pl.pallas_call(
        body,
        out_shape=jax.ShapeDtypeStruct((B, SQ, D_MODEL), jnp.float32),
        in_specs=[pl.BlockSpec(memory_space=pltpu.VMEM)] * 5,
        out_specs=pl.BlockSpec(memory_space=pltpu.VMEM),
        scratch_shapes=[
            pltpu.VMEM((2, NPAIR, CHUNK, 2 * DH), jnp.bfloat16),
            pltpu.VMEM((2, BH, CHUNK), jnp.float32),
            pltpu.VMEM((1, NPAIR, CHUNK, 2 * DH), jnp.float32),
            pltpu.VMEM((1, BH, CHUNK), jnp.float32),
            pltpu.VMEM((B, CHUNK, HQ * DH), jnp.float32),
            pltpu.VMEM((4, B, CHUNK, D_MODEL), jnp.bfloat16),
            pltpu.SemaphoreType.DMA((2,)),
            pltpu.SemaphoreType.DMA((2,)),
            pltpu.SemaphoreType.DMA((2,)),
            pltpu.SemaphoreType.DMA((2,)),
            pltpu.SemaphoreType.DMA((3,)),
            pltpu.SemaphoreType.DMA((4,)),
        ],
        compiler_params=pltpu.CompilerParams(collective_id=0),
    )(x, Wq2, K2, V2, Wo)


# device time: 42312 ns/iter; 1.6899x vs baseline; 1.0510x over previous
import jax
import jax.numpy as jnp
from jax import lax
from jax.experimental import pallas as pl
from jax.experimental.pallas import tpu as pltpu

N_DEV = 4
B, SQ, HQ, DH = 2, 512, 8, 64
SKV_LOC = 512
D_MODEL = 768
BH = B * HQ
BLK = 64
CHUNK = SQ // N_DEV
NPAIR = BH // 2
HPAIR = HQ // 2
NSLAB = NPAIR + 1


def kernel(x, Wq, K_ext, V_ext, Wo):
    K2 = K_ext.transpose(0, 2, 1, 3).reshape(BH, SKV_LOC, DH)
    V2 = V_ext.transpose(0, 2, 1, 3).reshape(BH, SKV_LOC, DH)
    Wq2 = Wq.reshape(D_MODEL, HPAIR, 2 * DH).transpose(1, 0, 2)

    def body(x_ref, wq_ref, k_ref, v_ref, wo_ref, out_ref,
             rs_o, p_o, ctx_s, ag,
             rso_send, rso_recv, ag_send, ag_recv):
        my = lax.axis_index("i")
        left = lax.rem(my + N_DEV - 1, N_DEV)
        right = lax.rem(my + 1, N_DEV)

        barrier = pltpu.get_barrier_semaphore()
        for nbr in (left, right):
            pl.semaphore_signal(barrier, inc=1, device_id=(nbr,),
                                device_id_type=pl.DeviceIdType.MESH)
        pl.semaphore_wait(barrier, 2)

        def compute_partial(c, o_dst, o_slot, o_dtype):
            qb = (lax.broadcasted_iota(jnp.int32, (CHUNK, SKV_LOC), 0)
                  + c * CHUNK) // BLK
            kb = (lax.broadcasted_iota(jnp.int32, (CHUNK, SKV_LOC), 1) // BLK
                  + my * (SKV_LOC // BLK))
            m = (qb == kb) | (kb == 0) | (((qb + kb) % 3) == 0)
            for b in range(B):
                xb = x_ref[b, pl.ds(c * CHUNK, CHUNK), :]
                for hp in range(HPAIR):
                    q2 = jnp.dot(xb, wq_ref[hp],
                                 preferred_element_type=jnp.float32)
                    for sub in range(2):
                        h = 2 * hp + sub
                        bh = b * HQ + h
                        pair, off = bh // 2, sub * DH
                        q = q2[:, off:off + DH]
                        s = lax.dot_general(
                            q, k_ref[bh], (((1,), (1,)), ((), ())),
                            preferred_element_type=jnp.float32,
                        ) * 0.125
                        w = jnp.where(m, jnp.exp(s), 0.0)
                        o = jnp.dot(w, v_ref[bh],
                                    preferred_element_type=jnp.float32)
                        o_dst[o_slot, pair, :, off:off + DH] = o.astype(o_dtype)
                        o_dst[o_slot, NPAIR, :, bh:bh + 1] = (
                            jnp.sum(w, axis=1, keepdims=True).astype(o_dtype))

        compute_partial(lax.rem(my + N_DEV - 1, N_DEV), rs_o, 0, jnp.bfloat16)
        for t in range(N_DEV - 1):
            ss, rs = t % 2, (t + 1) % 2
            ro = pltpu.make_async_remote_copy(
                src_ref=rs_o.at[ss], dst_ref=rs_o.at[rs],
                send_sem=rso_send.at[ss], recv_sem=rso_recv.at[rs],
                device_id=(right,), device_id_type=pl.DeviceIdType.MESH)
            ro.start()
            compute_partial(lax.rem(my + 2 * N_DEV - 2 - t, N_DEV),
                            p_o, 0, jnp.float32)
            ro.wait()
            rs_o[rs] = (rs_o[rs].astype(jnp.float32)
                        + p_o[0]).astype(jnp.bfloat16)
        fin = (N_DEV - 1) % 2

        r1 = []
        for b in range(B):
            for h in range(HQ):
                bh = b * HQ + h
                pair, off = bh // 2, (bh % 2) * DH
                lcol = rs_o[fin, NPAIR, :, bh:bh + 1].astype(jnp.float32)
                ctx_s[b, :, h * DH:(h + 1) * DH] = (
                    rs_o[fin, pair, :, off:off + DH].astype(jnp.float32)
                    / lcol)
            oc = jnp.dot(ctx_s[b], wo_ref[:, :],
                         preferred_element_type=jnp.float32)
            ag[0, b] = oc.astype(jnp.bfloat16)
            out_ref[b, pl.ds(my * CHUNK, CHUNK), :] = oc
            a_r = pltpu.make_async_remote_copy(
                src_ref=ag.at[0, b], dst_ref=ag.at[1, b],
                send_sem=ag_send.at[2 * b], recv_sem=ag_recv.at[2 * b],
                device_id=(right,), device_id_type=pl.DeviceIdType.MESH)
            a_l = pltpu.make_async_remote_copy(
                src_ref=ag.at[0, b], dst_ref=ag.at[2, b],
                send_sem=ag_send.at[2 * b + 1], recv_sem=ag_recv.at[2 * b + 1],
                device_id=(left,), device_id_type=pl.DeviceIdType.MESH)
            a_r.start()
            a_l.start()
            r1.append((a_r, a_l))
        for a_r, a_l in r1:
            a_r.wait()
            a_l.wait()
        fwd_r = pltpu.make_async_remote_copy(
            src_ref=ag.at[1, 0], dst_ref=ag.at[3, 0],
            send_sem=ag_send.at[4], recv_sem=ag_recv.at[4],
            device_id=(right,), device_id_type=pl.DeviceIdType.MESH)
        fwd_l = pltpu.make_async_remote_copy(
            src_ref=ag.at[2, 1], dst_ref=ag.at[3, 1],
            send_sem=ag_send.at[5], recv_sem=ag_recv.at[5],
            device_id=(left,), device_id_type=pl.DeviceIdType.MESH)
        fwd_r.start()
        fwd_l.start()
        for slot, origin in ((1, left), (2, right)):
            for b in range(B):
                out_ref[b, pl.ds(origin * CHUNK, CHUNK), :] = (
                    ag[slot, b].astype(jnp.float32))
        fwd_r.wait()
        fwd_l.wait()
        opp = lax.rem(my + 2, N_DEV)
        for b in range(B):
            out_ref[b, pl.ds(opp * CHUNK, CHUNK), :] = (
                ag[3, b].astype(jnp.float32))

    return pl.pallas_call(
        body,
        out_shape=jax.ShapeDtypeStruct((B, SQ, D_MODEL), jnp.float32),
        in_specs=[pl.BlockSpec(memory_space=pltpu.VMEM)] * 5,
        out_specs=pl.BlockSpec(memory_space=pltpu.VMEM),
        scratch_shapes=[
            pltpu.VMEM((2, NSLAB, CHUNK, 2 * DH), jnp.bfloat16),
            pltpu.VMEM((1, NSLAB, CHUNK, 2 * DH), jnp.float32),
            pltpu.VMEM((B, CHUNK, HQ * DH), jnp.float32),
            pltpu.VMEM((4, B, CHUNK, D_MODEL), jnp.bfloat16),
            pltpu.SemaphoreType.DMA((2,)),
            pltpu.SemaphoreType.DMA((2,)),
            pltpu.SemaphoreType.DMA((6,)),
            pltpu.SemaphoreType.DMA((6,)),
        ],
        compiler_params=pltpu.CompilerParams(collective_id=0),
    )(x, Wq2, K2, V2, Wo)
